# Initial kernel scaffold; baseline (speedup 1.0000x reference)
import jax
import jax.numpy as jnp
from jax.experimental import pallas as pl


def kernel(x, Wg, bg, We, be):
    raise NotImplementedError("write your pallas kernel here")


# speedup vs baseline: 7.0591x; 7.0591x over previous
import functools

import jax
import jax.numpy as jnp
from jax.experimental import pallas as pl
from jax.experimental.pallas import tpu as pltpu

B, S, D, E, K, O = 2, 2048, 2048, 8, 2, 2048
T = B * S
PREC = jax.lax.Precision.HIGHEST

TM_R = 1024
TM = 1024


def _router_body(x_ref, wg_ref, bg_ref, w_ref):
    logits = jnp.dot(x_ref[...], wg_ref[...], precision=PREC,
                     preferred_element_type=jnp.float32) + bg_ref[...]
    m = jnp.max(logits, axis=-1, keepdims=True)
    ex = jnp.exp(logits - m)
    p = ex / jnp.sum(ex, axis=-1, keepdims=True)
    lane = jax.lax.broadcasted_iota(jnp.int32, p.shape, 1)
    m1 = jnp.max(p, axis=-1, keepdims=True)
    i1 = jnp.min(jnp.where(p == m1, lane, E), axis=-1, keepdims=True)
    first1 = lane == i1
    p_rest = jnp.where(first1, -jnp.inf, p)
    m2 = jnp.max(p_rest, axis=-1, keepdims=True)
    i2 = jnp.min(jnp.where(p_rest == m2, lane, E), axis=-1, keepdims=True)
    first2 = lane == i2
    denom = m1 + m2 + 1e-9
    w = jnp.where(first1, p, 0.0) + jnp.where(first2, p, 0.0)
    w_ref[...] = w / denom


def _moe_body(x_ref, we_ref, w_ref, be_ref, out_ref):
    e = pl.program_id(1)
    y = jnp.dot(x_ref[...], we_ref[0], preferred_element_type=jnp.float32)
    w = w_ref[...]
    lane = jax.lax.broadcasted_iota(jnp.int32, w.shape, 1)
    wcol = jnp.sum(jnp.where(lane == e, w, 0.0), axis=-1, keepdims=True)
    y = y * wcol

    @pl.when(e == 0)
    def _():
        out_ref[...] = y + jnp.dot(w_ref[...], be_ref[...], precision=PREC,
                                   preferred_element_type=jnp.float32)

    @pl.when(e != 0)
    def _():
        out_ref[...] += y


@jax.jit
def kernel(x, Wg, bg, We, be):
    xf = x.reshape(T, D)
    w = pl.pallas_call(
        _router_body,
        grid=(T // TM_R,),
        in_specs=[
            pl.BlockSpec((TM_R, D), lambda i: (i, 0)),
            pl.BlockSpec((D, E), lambda i: (0, 0)),
            pl.BlockSpec((E,), lambda i: (0,)),
        ],
        out_specs=pl.BlockSpec((TM_R, E), lambda i: (i, 0)),
        out_shape=jax.ShapeDtypeStruct((T, E), jnp.float32),
    )(xf, Wg, bg)

    x_bf = xf.astype(jnp.bfloat16)
    We_bf = We.astype(jnp.bfloat16)
    out = pl.pallas_call(
        _moe_body,
        grid=(T // TM, E),
        in_specs=[
            pl.BlockSpec((TM, D), lambda i, e: (i, 0)),
            pl.BlockSpec((1, D, O), lambda i, e: (e, 0, 0)),
            pl.BlockSpec((TM, E), lambda i, e: (i, 0)),
            pl.BlockSpec((E, O), lambda i, e: (0, 0)),
        ],
        out_specs=pl.BlockSpec((TM, O), lambda i, e: (i, 0)),
        out_shape=jax.ShapeDtypeStruct((T, O), jnp.float32),
    )(x_bf, We_bf, w, be)
    return out.reshape(B, S, O)
